# Initial kernel scaffold; baseline (speedup 1.0000x reference)
import jax
import jax.numpy as jnp
from jax.experimental import pallas as pl


def kernel(x, adj, Wrel1, Wroot1, b1, p1, Wrel2, Wroot2, b2, p2, Wrel3, Wroot3, b3, p3, Wl1, bl1, Wl2, bl2, Wl3, bl3):
    raise NotImplementedError("write your pallas kernel here")


# speedup vs baseline: 7.2331x; 7.2331x over previous
import jax
import jax.numpy as jnp
from jax import lax
from jax.experimental import pallas as pl
from jax.experimental.pallas import tpu as pltpu
from jax.experimental.pallas import tpu_sc as plsc

N = 10000
E = 160000
H = 128
F_IN = 256
NC = 2
NS = 16
K1, K2, K3 = 8000, 6400, 5120
NPAD = 10240
BR = NPAD // 128

def _mesh():
    return plsc.VectorSubcoreMesh(
        core_axis_name="c", subcore_axis_name="s",
        num_cores=NC, num_subcores=NS)


ECH = 80
NCHK = E // NS // ECH
NHALF = NPAD // 2
TRASH = 512
AROWS = NHALF + TRASH
ZR = AROWS // NS
WR = NHALF // NS


def _edge_pass(c, s, x_hbm, out_hbm, zero_hbm, src_v, dst_v, dstm_v, buf,
               agg_s, sem):
    pltpu.sync_copy(zero_hbm.at[pl.ds(s * ZR, ZR)],
                    agg_s.at[pl.ds(s * ZR, ZR)])
    plsc.subcore_barrier()
    base = c * NHALF

    def chunk(j, carry):
        pltpu.async_copy(x_hbm.at[src_v.at[j]], buf, sem).wait()
        for g in range(ECH // 16):
            d = dst_v[j, pl.ds(g * 16, 16)]
            idx = d - base
            ok = (idx >= 0) & (idx < NHALF)
            trash = NHALF + (d & (TRASH - 1))
            dstm_v[pl.ds(g * 16, 16)] = jnp.where(ok, idx, trash)
        pltpu.sync_copy(buf, agg_s.at[dstm_v], add=True)
        return carry

    lax.fori_loop(0, NCHK, chunk, 0)
    plsc.subcore_barrier()
    pltpu.sync_copy(agg_s.at[pl.ds(s * WR, WR)],
                    out_hbm.at[pl.ds(c * NHALF + s * WR, WR)])
    plsc.subcore_barrier()


def _edge_l1_body(x0_hbm, x1_hbm, src_hbm, dst_hbm, zero_hbm,
                  a0_hbm, a1_hbm, src_v, dst_v, dstm_v, buf, agg_s, sem):
    c = lax.axis_index("c")
    s = lax.axis_index("s")
    pltpu.sync_copy(src_hbm.at[s], src_v)
    pltpu.sync_copy(dst_hbm.at[s], dst_v)
    _edge_pass(c, s, x0_hbm, a0_hbm, zero_hbm, src_v, dst_v, dstm_v, buf,
               agg_s, sem)
    _edge_pass(c, s, x1_hbm, a1_hbm, zero_hbm, src_v, dst_v, dstm_v, buf,
               agg_s, sem)


def _edge_l23_body(x_hbm, src_hbm, dst_hbm, zero_hbm, a_hbm,
                   src_v, dst_v, dstm_v, buf, agg_s, sem):
    c = lax.axis_index("c")
    s = lax.axis_index("s")
    pltpu.sync_copy(src_hbm.at[s], src_v)
    pltpu.sync_copy(dst_hbm.at[s], dst_v)
    _edge_pass(c, s, x_hbm, a_hbm, zero_hbm, src_v, dst_v, dstm_v, buf,
               agg_s, sem)


def _edge_scratch():
    return [
        pltpu.VMEM((NCHK, ECH), jnp.int32),
        pltpu.VMEM((NCHK, ECH), jnp.int32),
        pltpu.VMEM((ECH,), jnp.int32),
        pltpu.VMEM((ECH, H), jnp.float32),
        pltpu.VMEM_SHARED((AROWS, H), jnp.float32),
        pltpu.SemaphoreType.DMA,
    ]


def _edge_l1(x0, x1, src3, dst3, zeros):
    f = pl.kernel(
        _edge_l1_body,
        out_type=(jax.ShapeDtypeStruct((NPAD, H), jnp.float32),
                  jax.ShapeDtypeStruct((NPAD, H), jnp.float32)),
        mesh=_mesh(),
        scratch_types=_edge_scratch(),
    )
    return f(x0, x1, src3, dst3, zeros)


def _edge_l23(x, src3, dst3, zeros):
    f = pl.kernel(
        _edge_l23_body,
        out_type=jax.ShapeDtypeStruct((NPAD, H), jnp.float32),
        mesh=_mesh(),
        scratch_types=_edge_scratch(),
    )
    return f(x, src3, dst3, zeros)


def _monokey(score):
    s = score + 0.0
    u = lax.bitcast_convert_type(s, jnp.uint32)
    neg = (u >> 31) == jnp.uint32(1)
    return jnp.where(neg, ~u, u | jnp.uint32(0x80000000))


def _combine1_kernel(a0_ref, a1_ref, x_ref, wrelt_ref,
                     wroott_ref, b_ref, p_ref, h_ref, score_ref, key_ref):
    agg = jnp.concatenate([a0_ref[0:N, :], a1_ref[0:N, :]], axis=1)
    a = jnp.dot(agg, wrelt_ref[...], preferred_element_type=jnp.float32)
    h = jnp.maximum(
        a + b_ref[...] + jnp.dot(x_ref[...], wroott_ref[...],
                                 preferred_element_type=jnp.float32), 0.0)
    h_ref[...] = h
    p = p_ref[...]
    nrm = jnp.sqrt(jnp.sum(p * p)) + 1e-16
    sr = jnp.dot(h, jnp.reshape(p, (H, 1)),
                 preferred_element_type=jnp.float32)
    score = jnp.tanh(sr / nrm)
    score_ref[...] = score
    key_ref[...] = _monokey(score)


def _combine23_kernel(a_ref, hm_ref, wrelt_ref, wroott_ref,
                      b_ref, p_ref, alive_ref, h_ref, score_ref, key_ref):
    agg = a_ref[0:N, :]
    h = jnp.maximum(
        jnp.dot(agg, wrelt_ref[...], preferred_element_type=jnp.float32)
        + b_ref[...]
        + jnp.dot(hm_ref[...], wroott_ref[...],
                  preferred_element_type=jnp.float32), 0.0)
    h_ref[...] = h
    p = p_ref[...]
    nrm = jnp.sqrt(jnp.sum(p * p)) + 1e-16
    sr = jnp.dot(h, jnp.reshape(p, (H, 1)),
                 preferred_element_type=jnp.float32)
    score = jnp.tanh(sr / nrm)
    score_ref[...] = score
    key_ref[...] = jnp.where(alive_ref[...] > 0.0, _monokey(score),
                             jnp.uint32(0))


def _bisect_theta(u2d, eq2, m):
    def body(_, lohi):
        lo, hi = lohi
        mid = lo + ((hi - lo) >> 1)
        c = jnp.sum((eq2 & (u2d > mid)).astype(jnp.int32))
        sel = c <= (m - 1)
        return (jnp.where(sel, lo, mid + 1), jnp.where(sel, mid, hi))
    lo, hi = lax.fori_loop(
        0, 32, body, (jnp.uint32(0), jnp.uint32(0xFFFFFFFF)))
    return hi


def _make_pool_kernel(k, nlevels):
    def kern(*refs):
        h_ref = refs[0]
        score_ref = refs[1]
        keyn = refs[2:2 + nlevels]
        key2 = refs[2 + nlevels:2 + 2 * nlevels]
        hm_ref, alive_ref, ro_ref = refs[2 + 2 * nlevels:]

        eq2 = jnp.ones((BR, 128), bool)
        m = jnp.int32(k)
        thetas = []
        for u_ref in key2:
            u2d = u_ref[...]
            theta = _bisect_theta(u2d, eq2, m)
            gt = eq2 & (u2d > theta)
            m = m - jnp.sum(gt.astype(jnp.int32))
            eq2 = eq2 & (u2d == theta)
            thetas.append(theta)

        idx2 = (lax.broadcasted_iota(jnp.int32, (BR, 128), 0) * 128
                + lax.broadcasted_iota(jnp.int32, (BR, 128), 1))

        def ibody(_, lohi):
            lo, hi = lohi
            mid = lo + ((hi - lo) >> 1)
            c = jnp.sum((eq2 & (idx2 < mid)).astype(jnp.int32))
            sel = c >= m
            return (jnp.where(sel, lo, mid + 1), jnp.where(sel, mid, hi))
        _, cut = lax.fori_loop(0, 15, ibody, (jnp.int32(0), jnp.int32(16384)))

        Mn = jnp.zeros((N, 1), bool)
        eqn = jnp.ones((N, 1), bool)
        for u_ref, theta in zip(keyn, thetas):
            un = u_ref[...]
            Mn = Mn | (eqn & (un > theta))
            eqn = eqn & (un == theta)
        idxn = lax.broadcasted_iota(jnp.int32, (N, 1), 0)
        Mn = Mn | (eqn & (idxn < cut))

        hm = (h_ref[...] * score_ref[...]) * Mn.astype(jnp.float32)
        hm_ref[...] = hm
        alive_ref[...] = Mn.astype(jnp.float32)
        rmax = jnp.max(jnp.where(Mn, hm, -jnp.inf), axis=0, keepdims=True)
        rmean = jnp.sum(hm, axis=0, keepdims=True) / k
        ro_ref[...] = jnp.concatenate([rmax, rmean], axis=1)
    return kern


def _pool(h, score, keyn_list, key2_list, k):
    nlevels = len(keyn_list)
    f = pl.pallas_call(
        _make_pool_kernel(k, nlevels),
        out_shape=(jax.ShapeDtypeStruct((N, H), jnp.float32),
                   jax.ShapeDtypeStruct((N, 1), jnp.float32),
                   jax.ShapeDtypeStruct((1, 2 * H), jnp.float32)),
    )
    return f(h, score, *keyn_list, *key2_list)


def _mlp_kernel(ro1_ref, ro2_ref, ro3_ref, wl1t_ref, bl1_ref, wl2t_ref,
                bl2_ref, wl3t_ref, bl3_ref, logits_ref, prob_ref, yhat_ref):
    z = ro1_ref[...] + ro2_ref[...] + ro3_ref[...]
    z = jnp.maximum(jnp.dot(z, wl1t_ref[...],
                            preferred_element_type=jnp.float32)
                    + bl1_ref[...], 0.0)
    z = jnp.maximum(jnp.dot(z, wl2t_ref[...],
                            preferred_element_type=jnp.float32)
                    + bl2_ref[...], 0.0)
    logits = jnp.dot(z, wl3t_ref[...],
                     preferred_element_type=jnp.float32) + bl3_ref[...]
    logits_ref[...] = logits
    mx = jnp.max(logits, axis=1, keepdims=True)
    ex = jnp.exp(logits - mx)
    prob_ref[...] = ex / jnp.sum(ex, axis=1, keepdims=True)
    yhat_ref[...] = (logits[:, 1:2] > logits[:, 0:1]).astype(jnp.int32)


def _mlp(ro1, ro2, ro3, wl1t, bl1, wl2t, bl2, wl3t, bl3):
    f = pl.pallas_call(
        _mlp_kernel,
        out_shape=(jax.ShapeDtypeStruct((1, 2), jnp.float32),
                   jax.ShapeDtypeStruct((1, 2), jnp.float32),
                   jax.ShapeDtypeStruct((1, 1), jnp.int32)),
    )
    return f(ro1, ro2, ro3, wl1t, bl1, wl2t, bl2, wl3t, bl3)


def _combine1(a0, a1, x, wrelt, wroott, b, p):
    f = pl.pallas_call(
        _combine1_kernel,
        out_shape=(jax.ShapeDtypeStruct((N, H), jnp.float32),
                   jax.ShapeDtypeStruct((N, 1), jnp.float32),
                   jax.ShapeDtypeStruct((N, 1), jnp.uint32)),
    )
    return f(a0, a1, x, wrelt, wroott, b, p)


def _combine23(a, hm, wrelt, wroott, b, p, alive):
    f = pl.pallas_call(
        _combine23_kernel,
        out_shape=(jax.ShapeDtypeStruct((N, H), jnp.float32),
                   jax.ShapeDtypeStruct((N, 1), jnp.float32),
                   jax.ShapeDtypeStruct((N, 1), jnp.uint32)),
    )
    return f(a, hm, wrelt, wroott, b, p, alive)


def _to2d(key):
    v = jnp.pad(jnp.reshape(key, (N,)), (0, NPAD - N))
    return jnp.reshape(v, (BR, 128))


def kernel(x, adj, Wrel1, Wroot1, b1, p1, Wrel2, Wroot2, b2, p2,
           Wrel3, Wroot3, b3, p3, Wl1, bl1, Wl2, bl2, Wl3, bl3):
    src = adj[0].astype(jnp.int32)
    dst = adj[1].astype(jnp.int32)
    zeros = jnp.zeros((AROWS, H), jnp.float32)

    src16 = jnp.reshape(src, (NS, NCHK, ECH))
    dst16 = jnp.reshape(dst, (NS, NCHK, ECH))

    a0, a1 = _edge_l1(x[:, 0:H], x[:, H:F_IN], src16, dst16, zeros)
    h, score, key1 = _combine1(a0, a1, x, Wrel1.T, Wroot1.T,
                               jnp.reshape(b1, (1, H)), jnp.reshape(p1, (1, H)))
    hm, alive, ro1 = _pool(h, score, [key1], [_to2d(key1)], K1)

    a = _edge_l23(hm, src16, dst16, zeros)
    h, score, key2 = _combine23(a, hm, Wrel2.T, Wroot2.T,
                                jnp.reshape(b2, (1, H)),
                                jnp.reshape(p2, (1, H)), alive)
    hm, alive, ro2 = _pool(h, score, [key2, key1],
                           [_to2d(key2), _to2d(key1)], K2)

    a = _edge_l23(hm, src16, dst16, zeros)
    h, score, key3 = _combine23(a, hm, Wrel3.T, Wroot3.T,
                                jnp.reshape(b3, (1, H)),
                                jnp.reshape(p3, (1, H)), alive)
    hm, alive, ro3 = _pool(h, score, [key3, key2, key1],
                           [_to2d(key3), _to2d(key2), _to2d(key1)], K3)

    logits, prob, yhat = _mlp(ro1, ro2, ro3, Wl1.T, jnp.reshape(bl1, (1, 128)),
                              Wl2.T, jnp.reshape(bl2, (1, 64)),
                              Wl3.T, jnp.reshape(bl3, (1, 2)))
    return (logits, prob, yhat)


# speedup vs baseline: 11.4537x; 1.5835x over previous
import jax
import jax.numpy as jnp
from jax import lax
from jax.experimental import pallas as pl
from jax.experimental.pallas import tpu as pltpu
from jax.experimental.pallas import tpu_sc as plsc

N = 10000
E = 160000
H = 128
F_IN = 256
NC = 2
NS = 16
K1, K2, K3 = 8000, 6400, 5120
NPAD = 10240
BR = NPAD // 128

def _mesh():
    return plsc.VectorSubcoreMesh(
        core_axis_name="c", subcore_axis_name="s",
        num_cores=NC, num_subcores=NS)


ECH = 80
NCHK = E // NS // ECH
NHALF = NPAD // 2
TRASH = 256
AROWS = NHALF + TRASH
ZR = AROWS // NS
WR = NHALF // NS


def _edge_pass(c, s, x_hbm, out_hbm, zero_hbm, src_v, dst_v, dstm_a, dstm_b,
               buf_a, buf_b, agg_s, sem_a, sem_b):
    pltpu.sync_copy(zero_hbm.at[pl.ds(s * ZR, ZR)],
                    agg_s.at[pl.ds(s * ZR, ZR)])
    plsc.subcore_barrier()
    base = c * NHALF

    def remap(j, dstm):
        for g in range(ECH // 16):
            d = dst_v[j, pl.ds(g * 16, 16)]
            idx = d - base
            ok = (idx >= 0) & (idx < NHALF)
            trash = NHALF + (d & (TRASH - 1))
            dstm[pl.ds(g * 16, 16)] = jnp.where(ok, idx, trash)

    def drain(buf, sem):
        pltpu.make_async_copy(x_hbm.at[src_v.at[0]], buf, sem).wait()

    pltpu.async_copy(x_hbm.at[src_v.at[0]], buf_a, sem_a)

    def pair(i, carry):
        j0 = 2 * i
        pltpu.async_copy(x_hbm.at[src_v.at[j0 + 1]], buf_b, sem_b)
        remap(j0, dstm_a)
        drain(buf_a, sem_a)
        pltpu.sync_copy(buf_a, agg_s.at[dstm_a], add=True)
        pltpu.async_copy(x_hbm.at[src_v.at[j0 + 2]], buf_a, sem_a)
        remap(j0 + 1, dstm_b)
        drain(buf_b, sem_b)
        pltpu.sync_copy(buf_b, agg_s.at[dstm_b], add=True)
        return carry

    lax.fori_loop(0, (NCHK - 1) // 2, pair, 0)
    remap(NCHK - 1, dstm_a)
    drain(buf_a, sem_a)
    pltpu.sync_copy(buf_a, agg_s.at[dstm_a], add=True)
    plsc.subcore_barrier()
    pltpu.sync_copy(agg_s.at[pl.ds(s * WR, WR)],
                    out_hbm.at[pl.ds(c * NHALF + s * WR, WR)])
    plsc.subcore_barrier()


def _edge_l1_body(x0_hbm, x1_hbm, src_hbm, dst_hbm, zero_hbm,
                  a0_hbm, a1_hbm, src_v, dst_v, dstm_a, dstm_b, buf_a, buf_b,
                  agg_s, sem_a, sem_b):
    c = lax.axis_index("c")
    s = lax.axis_index("s")
    pltpu.sync_copy(src_hbm.at[s], src_v)
    pltpu.sync_copy(dst_hbm.at[s], dst_v)
    _edge_pass(c, s, x0_hbm, a0_hbm, zero_hbm, src_v, dst_v, dstm_a, dstm_b,
               buf_a, buf_b, agg_s, sem_a, sem_b)
    _edge_pass(c, s, x1_hbm, a1_hbm, zero_hbm, src_v, dst_v, dstm_a, dstm_b,
               buf_a, buf_b, agg_s, sem_a, sem_b)


def _edge_l23_body(x_hbm, src_hbm, dst_hbm, zero_hbm, a_hbm,
                   src_v, dst_v, dstm_a, dstm_b, buf_a, buf_b,
                   agg_s, sem_a, sem_b):
    c = lax.axis_index("c")
    s = lax.axis_index("s")
    pltpu.sync_copy(src_hbm.at[s], src_v)
    pltpu.sync_copy(dst_hbm.at[s], dst_v)
    _edge_pass(c, s, x_hbm, a_hbm, zero_hbm, src_v, dst_v, dstm_a, dstm_b,
               buf_a, buf_b, agg_s, sem_a, sem_b)


def _edge_scratch():
    return [
        pltpu.VMEM((NCHK, ECH), jnp.int32),
        pltpu.VMEM((NCHK, ECH), jnp.int32),
        pltpu.VMEM((ECH,), jnp.int32),
        pltpu.VMEM((ECH,), jnp.int32),
        pltpu.VMEM((ECH, H), jnp.float32),
        pltpu.VMEM((ECH, H), jnp.float32),
        pltpu.VMEM_SHARED((AROWS, H), jnp.float32),
        pltpu.SemaphoreType.DMA,
        pltpu.SemaphoreType.DMA,
    ]


def _edge_l1(x0, x1, src3, dst3, zeros):
    f = pl.kernel(
        _edge_l1_body,
        out_type=(jax.ShapeDtypeStruct((NPAD, H), jnp.float32),
                  jax.ShapeDtypeStruct((NPAD, H), jnp.float32)),
        mesh=_mesh(),
        scratch_types=_edge_scratch(),
    )
    return f(x0, x1, src3, dst3, zeros)


def _edge_l23(x, src3, dst3, zeros):
    f = pl.kernel(
        _edge_l23_body,
        out_type=jax.ShapeDtypeStruct((NPAD, H), jnp.float32),
        mesh=_mesh(),
        scratch_types=_edge_scratch(),
    )
    return f(x, src3, dst3, zeros)


def _monokey(score):
    s = score + 0.0
    u = lax.bitcast_convert_type(s, jnp.uint32)
    neg = (u >> 31) == jnp.uint32(1)
    return jnp.where(neg, ~u, u | jnp.uint32(0x80000000))


def _combine1_kernel(a0_ref, a1_ref, x_ref, wrelt_ref,
                     wroott_ref, b_ref, p_ref, h_ref, score_ref, key_ref):
    agg = jnp.concatenate([a0_ref[0:N, :], a1_ref[0:N, :]], axis=1)
    a = jnp.dot(agg, wrelt_ref[...], preferred_element_type=jnp.float32)
    h = jnp.maximum(
        a + b_ref[...] + jnp.dot(x_ref[...], wroott_ref[...],
                                 preferred_element_type=jnp.float32), 0.0)
    h_ref[...] = h
    p = p_ref[...]
    nrm = jnp.sqrt(jnp.sum(p * p)) + 1e-16
    sr = jnp.dot(h, jnp.reshape(p, (H, 1)),
                 preferred_element_type=jnp.float32)
    score = jnp.tanh(sr / nrm)
    score_ref[...] = score
    key_ref[...] = _monokey(score)


def _combine23_kernel(a_ref, hm_ref, wrelt_ref, wroott_ref,
                      b_ref, p_ref, alive_ref, h_ref, score_ref, key_ref):
    agg = a_ref[0:N, :]
    h = jnp.maximum(
        jnp.dot(agg, wrelt_ref[...], preferred_element_type=jnp.float32)
        + b_ref[...]
        + jnp.dot(hm_ref[...], wroott_ref[...],
                  preferred_element_type=jnp.float32), 0.0)
    h_ref[...] = h
    p = p_ref[...]
    nrm = jnp.sqrt(jnp.sum(p * p)) + 1e-16
    sr = jnp.dot(h, jnp.reshape(p, (H, 1)),
                 preferred_element_type=jnp.float32)
    score = jnp.tanh(sr / nrm)
    score_ref[...] = score
    key_ref[...] = jnp.where(alive_ref[...] > 0.0, _monokey(score),
                             jnp.uint32(0))


def _bisect_theta(u2d, eq2, m):
    def body(_, lohi):
        lo, hi = lohi
        mid = lo + ((hi - lo) >> 1)
        c = jnp.sum((eq2 & (u2d > mid)).astype(jnp.int32))
        sel = c <= (m - 1)
        return (jnp.where(sel, lo, mid + 1), jnp.where(sel, mid, hi))
    lo, hi = lax.fori_loop(
        0, 32, body, (jnp.uint32(0), jnp.uint32(0xFFFFFFFF)))
    return hi


def _make_pool_kernel(k, nlevels):
    def kern(*refs):
        h_ref = refs[0]
        score_ref = refs[1]
        keyn = refs[2:2 + nlevels]
        key2 = refs[2 + nlevels:2 + 2 * nlevels]
        hm_ref, alive_ref, ro_ref = refs[2 + 2 * nlevels:]

        eq2 = jnp.ones((BR, 128), bool)
        m = jnp.int32(k)
        thetas = []
        for u_ref in key2:
            u2d = u_ref[...]
            theta = _bisect_theta(u2d, eq2, m)
            gt = eq2 & (u2d > theta)
            m = m - jnp.sum(gt.astype(jnp.int32))
            eq2 = eq2 & (u2d == theta)
            thetas.append(theta)

        idx2 = (lax.broadcasted_iota(jnp.int32, (BR, 128), 0) * 128
                + lax.broadcasted_iota(jnp.int32, (BR, 128), 1))

        def ibody(_, lohi):
            lo, hi = lohi
            mid = lo + ((hi - lo) >> 1)
            c = jnp.sum((eq2 & (idx2 < mid)).astype(jnp.int32))
            sel = c >= m
            return (jnp.where(sel, lo, mid + 1), jnp.where(sel, mid, hi))
        _, cut = lax.fori_loop(0, 15, ibody, (jnp.int32(0), jnp.int32(16384)))

        Mn = jnp.zeros((N, 1), bool)
        eqn = jnp.ones((N, 1), bool)
        for u_ref, theta in zip(keyn, thetas):
            un = u_ref[...]
            Mn = Mn | (eqn & (un > theta))
            eqn = eqn & (un == theta)
        idxn = lax.broadcasted_iota(jnp.int32, (N, 1), 0)
        Mn = Mn | (eqn & (idxn < cut))

        hm = (h_ref[...] * score_ref[...]) * Mn.astype(jnp.float32)
        hm_ref[...] = hm
        alive_ref[...] = Mn.astype(jnp.float32)
        rmax = jnp.max(jnp.where(Mn, hm, -jnp.inf), axis=0, keepdims=True)
        rmean = jnp.sum(hm, axis=0, keepdims=True) / k
        ro_ref[...] = jnp.concatenate([rmax, rmean], axis=1)
    return kern


def _pool(h, score, keyn_list, key2_list, k):
    nlevels = len(keyn_list)
    f = pl.pallas_call(
        _make_pool_kernel(k, nlevels),
        out_shape=(jax.ShapeDtypeStruct((N, H), jnp.float32),
                   jax.ShapeDtypeStruct((N, 1), jnp.float32),
                   jax.ShapeDtypeStruct((1, 2 * H), jnp.float32)),
    )
    return f(h, score, *keyn_list, *key2_list)


def _mlp_kernel(ro1_ref, ro2_ref, ro3_ref, wl1t_ref, bl1_ref, wl2t_ref,
                bl2_ref, wl3t_ref, bl3_ref, logits_ref, prob_ref, yhat_ref):
    z = ro1_ref[...] + ro2_ref[...] + ro3_ref[...]
    z = jnp.maximum(jnp.dot(z, wl1t_ref[...],
                            preferred_element_type=jnp.float32)
                    + bl1_ref[...], 0.0)
    z = jnp.maximum(jnp.dot(z, wl2t_ref[...],
                            preferred_element_type=jnp.float32)
                    + bl2_ref[...], 0.0)
    logits = jnp.dot(z, wl3t_ref[...],
                     preferred_element_type=jnp.float32) + bl3_ref[...]
    logits_ref[...] = logits
    mx = jnp.max(logits, axis=1, keepdims=True)
    ex = jnp.exp(logits - mx)
    prob_ref[...] = ex / jnp.sum(ex, axis=1, keepdims=True)
    yhat_ref[...] = (logits[:, 1:2] > logits[:, 0:1]).astype(jnp.int32)


def _mlp(ro1, ro2, ro3, wl1t, bl1, wl2t, bl2, wl3t, bl3):
    f = pl.pallas_call(
        _mlp_kernel,
        out_shape=(jax.ShapeDtypeStruct((1, 2), jnp.float32),
                   jax.ShapeDtypeStruct((1, 2), jnp.float32),
                   jax.ShapeDtypeStruct((1, 1), jnp.int32)),
    )
    return f(ro1, ro2, ro3, wl1t, bl1, wl2t, bl2, wl3t, bl3)


def _combine1(a0, a1, x, wrelt, wroott, b, p):
    f = pl.pallas_call(
        _combine1_kernel,
        out_shape=(jax.ShapeDtypeStruct((N, H), jnp.float32),
                   jax.ShapeDtypeStruct((N, 1), jnp.float32),
                   jax.ShapeDtypeStruct((N, 1), jnp.uint32)),
    )
    return f(a0, a1, x, wrelt, wroott, b, p)


def _combine23(a, hm, wrelt, wroott, b, p, alive):
    f = pl.pallas_call(
        _combine23_kernel,
        out_shape=(jax.ShapeDtypeStruct((N, H), jnp.float32),
                   jax.ShapeDtypeStruct((N, 1), jnp.float32),
                   jax.ShapeDtypeStruct((N, 1), jnp.uint32)),
    )
    return f(a, hm, wrelt, wroott, b, p, alive)


def _to2d(key):
    v = jnp.pad(jnp.reshape(key, (N,)), (0, NPAD - N))
    return jnp.reshape(v, (BR, 128))


def kernel(x, adj, Wrel1, Wroot1, b1, p1, Wrel2, Wroot2, b2, p2,
           Wrel3, Wroot3, b3, p3, Wl1, bl1, Wl2, bl2, Wl3, bl3):
    src = adj[0].astype(jnp.int32)
    dst = adj[1].astype(jnp.int32)
    zeros = jnp.zeros((AROWS, H), jnp.float32)

    src16 = jnp.reshape(src, (NS, NCHK, ECH))
    dst16 = jnp.reshape(dst, (NS, NCHK, ECH))

    a0, a1 = _edge_l1(x[:, 0:H], x[:, H:F_IN], src16, dst16, zeros)
    h, score, key1 = _combine1(a0, a1, x, Wrel1.T, Wroot1.T,
                               jnp.reshape(b1, (1, H)), jnp.reshape(p1, (1, H)))
    hm, alive, ro1 = _pool(h, score, [key1], [_to2d(key1)], K1)

    a = _edge_l23(hm, src16, dst16, zeros)
    h, score, key2 = _combine23(a, hm, Wrel2.T, Wroot2.T,
                                jnp.reshape(b2, (1, H)),
                                jnp.reshape(p2, (1, H)), alive)
    hm, alive, ro2 = _pool(h, score, [key2, key1],
                           [_to2d(key2), _to2d(key1)], K2)

    a = _edge_l23(hm, src16, dst16, zeros)
    h, score, key3 = _combine23(a, hm, Wrel3.T, Wroot3.T,
                                jnp.reshape(b3, (1, H)),
                                jnp.reshape(p3, (1, H)), alive)
    hm, alive, ro3 = _pool(h, score, [key3, key2, key1],
                           [_to2d(key3), _to2d(key2), _to2d(key1)], K3)

    logits, prob, yhat = _mlp(ro1, ro2, ro3, Wl1.T, jnp.reshape(bl1, (1, 128)),
                              Wl2.T, jnp.reshape(bl2, (1, 64)),
                              Wl3.T, jnp.reshape(bl3, (1, 2)))
    return (logits, prob, yhat)


# speedup vs baseline: 11.6569x; 1.0177x over previous
import jax
import jax.numpy as jnp
from jax import lax
from jax.experimental import pallas as pl
from jax.experimental.pallas import tpu as pltpu
from jax.experimental.pallas import tpu_sc as plsc

N = 10000
E = 160000
H = 128
F_IN = 256
NC = 2
NS = 16
K1, K2, K3 = 8000, 6400, 5120
NPAD = 10240
BR = NPAD // 128

def _mesh():
    return plsc.VectorSubcoreMesh(
        core_axis_name="c", subcore_axis_name="s",
        num_cores=NC, num_subcores=NS)


ECH = 80
NCHK = E // NS // ECH
NHALF = NPAD // 2
TRASH = 256
AROWS = NHALF + TRASH
ZR = AROWS // NS
WR = NHALF // NS


def _edge_pass(c, s, x_hbm, out_hbm, zero_hbm, src_v, dst_v, dstm, bufs,
               agg_s, gsems, ssems):
    pltpu.sync_copy(zero_hbm.at[pl.ds(s * ZR, ZR)],
                    agg_s.at[pl.ds(s * ZR, ZR)])
    plsc.subcore_barrier()
    base = c * NHALF

    def remap(j, dref):
        for g in range(ECH // 16):
            d = dst_v[j, pl.ds(g * 16, 16)]
            idx = d - base
            ok = (idx >= 0) & (idx < NHALF)
            trash = NHALF + (d & (TRASH - 1))
            dref[pl.ds(g * 16, 16)] = jnp.where(ok, idx, trash)

    def gfire(sl, j):
        pltpu.async_copy(x_hbm.at[src_v.at[j]], bufs[sl], gsems[sl])

    def gwait(sl, j):
        pltpu.make_async_copy(x_hbm.at[src_v.at[j]], bufs[sl],
                              gsems[sl]).wait()

    def sfire(sl):
        pltpu.async_copy(bufs[sl], agg_s.at[dstm[sl]], ssems[sl], add=True)

    def swait(sl):
        pltpu.make_async_copy(bufs[sl], agg_s.at[dstm[sl]], ssems[sl]).wait()

    gfire(0, 0)
    gfire(1, 1)

    def group(i, carry):
        q0 = 4 * i
        for sl in range(4):
            q = q0 + sl
            sl2 = (sl + 2) % 4
            gwait(sl, q)
            remap(q, dstm[sl])
            sfire(sl)

            @pl.when(jnp.logical_and(q >= 2, q + 2 < NCHK))
            def _():
                swait(sl2)

            @pl.when(q + 2 < NCHK)
            def _():
                gfire(sl2, q + 2)
        return carry

    lax.fori_loop(0, NCHK // 4, group, 0)
    gwait(0, NCHK - 1)
    remap(NCHK - 1, dstm[0])
    sfire(0)
    for sl in range(4):
        swait(sl)
    plsc.subcore_barrier()
    pltpu.sync_copy(agg_s.at[pl.ds(s * WR, WR)],
                    out_hbm.at[pl.ds(c * NHALF + s * WR, WR)])
    plsc.subcore_barrier()


def _edge_l1_body(x0_hbm, x1_hbm, src_hbm, dst_hbm, zero_hbm,
                  a0_hbm, a1_hbm, src_v, dst_v,
                  dstm0, dstm1, dstm2, dstm3, buf0, buf1, buf2, buf3,
                  agg_s, g0, g1, g2, g3, s0, s1, s2, s3):
    c = lax.axis_index("c")
    s = lax.axis_index("s")
    pltpu.sync_copy(src_hbm.at[s], src_v)
    pltpu.sync_copy(dst_hbm.at[s], dst_v)
    dstm = (dstm0, dstm1, dstm2, dstm3)
    bufs = (buf0, buf1, buf2, buf3)
    gsems = (g0, g1, g2, g3)
    ssems = (s0, s1, s2, s3)
    _edge_pass(c, s, x0_hbm, a0_hbm, zero_hbm, src_v, dst_v, dstm, bufs,
               agg_s, gsems, ssems)
    _edge_pass(c, s, x1_hbm, a1_hbm, zero_hbm, src_v, dst_v, dstm, bufs,
               agg_s, gsems, ssems)


def _edge_l23_body(x_hbm, src_hbm, dst_hbm, zero_hbm, a_hbm,
                   src_v, dst_v,
                   dstm0, dstm1, dstm2, dstm3, buf0, buf1, buf2, buf3,
                   agg_s, g0, g1, g2, g3, s0, s1, s2, s3):
    c = lax.axis_index("c")
    s = lax.axis_index("s")
    pltpu.sync_copy(src_hbm.at[s], src_v)
    pltpu.sync_copy(dst_hbm.at[s], dst_v)
    dstm = (dstm0, dstm1, dstm2, dstm3)
    bufs = (buf0, buf1, buf2, buf3)
    gsems = (g0, g1, g2, g3)
    ssems = (s0, s1, s2, s3)
    _edge_pass(c, s, x_hbm, a_hbm, zero_hbm, src_v, dst_v, dstm, bufs,
               agg_s, gsems, ssems)


def _edge_scratch():
    return ([
        pltpu.VMEM((NCHK, ECH), jnp.int32),
        pltpu.VMEM((NCHK, ECH), jnp.int32),
    ] + [pltpu.VMEM((ECH,), jnp.int32) for _ in range(4)]
      + [pltpu.VMEM((ECH, H), jnp.float32) for _ in range(4)]
      + [pltpu.VMEM_SHARED((AROWS, H), jnp.float32)]
      + [pltpu.SemaphoreType.DMA for _ in range(8)])


def _edge_l1(x0, x1, src3, dst3, zeros):
    f = pl.kernel(
        _edge_l1_body,
        out_type=(jax.ShapeDtypeStruct((NPAD, H), jnp.float32),
                  jax.ShapeDtypeStruct((NPAD, H), jnp.float32)),
        mesh=_mesh(),
        scratch_types=_edge_scratch(),
    )
    return f(x0, x1, src3, dst3, zeros)


def _edge_l23(x, src3, dst3, zeros):
    f = pl.kernel(
        _edge_l23_body,
        out_type=jax.ShapeDtypeStruct((NPAD, H), jnp.float32),
        mesh=_mesh(),
        scratch_types=_edge_scratch(),
    )
    return f(x, src3, dst3, zeros)


def _monokey(score):
    s = score + 0.0
    u = lax.bitcast_convert_type(s, jnp.uint32)
    neg = (u >> 31) == jnp.uint32(1)
    return jnp.where(neg, ~u, u | jnp.uint32(0x80000000))


def _combine1_kernel(a0_ref, a1_ref, x_ref, wrelt_ref,
                     wroott_ref, b_ref, p_ref, h_ref, score_ref, key_ref):
    agg = jnp.concatenate([a0_ref[0:N, :], a1_ref[0:N, :]], axis=1)
    a = jnp.dot(agg, wrelt_ref[...], preferred_element_type=jnp.float32)
    h = jnp.maximum(
        a + b_ref[...] + jnp.dot(x_ref[...], wroott_ref[...],
                                 preferred_element_type=jnp.float32), 0.0)
    h_ref[...] = h
    p = p_ref[...]
    nrm = jnp.sqrt(jnp.sum(p * p)) + 1e-16
    sr = jnp.dot(h, jnp.reshape(p, (H, 1)),
                 preferred_element_type=jnp.float32)
    score = jnp.tanh(sr / nrm)
    score_ref[...] = score
    key_ref[...] = _monokey(score)


def _combine23_kernel(a_ref, hm_ref, wrelt_ref, wroott_ref,
                      b_ref, p_ref, alive_ref, h_ref, score_ref, key_ref):
    agg = a_ref[0:N, :]
    h = jnp.maximum(
        jnp.dot(agg, wrelt_ref[...], preferred_element_type=jnp.float32)
        + b_ref[...]
        + jnp.dot(hm_ref[...], wroott_ref[...],
                  preferred_element_type=jnp.float32), 0.0)
    h_ref[...] = h
    p = p_ref[...]
    nrm = jnp.sqrt(jnp.sum(p * p)) + 1e-16
    sr = jnp.dot(h, jnp.reshape(p, (H, 1)),
                 preferred_element_type=jnp.float32)
    score = jnp.tanh(sr / nrm)
    score_ref[...] = score
    key_ref[...] = jnp.where(alive_ref[...] > 0.0, _monokey(score),
                             jnp.uint32(0))


def _bisect_theta(u2d, eq2, m):
    def body(_, lohi):
        lo, hi = lohi
        mid = lo + ((hi - lo) >> 1)
        c = jnp.sum((eq2 & (u2d > mid)).astype(jnp.int32))
        sel = c <= (m - 1)
        return (jnp.where(sel, lo, mid + 1), jnp.where(sel, mid, hi))
    lo, hi = lax.fori_loop(
        0, 32, body, (jnp.uint32(0), jnp.uint32(0xFFFFFFFF)))
    return hi


def _make_pool_kernel(k, nlevels):
    def kern(*refs):
        h_ref = refs[0]
        score_ref = refs[1]
        keyn = refs[2:2 + nlevels]
        key2 = refs[2 + nlevels:2 + 2 * nlevels]
        hm_ref, alive_ref, ro_ref = refs[2 + 2 * nlevels:]

        eq2 = jnp.ones((BR, 128), bool)
        m = jnp.int32(k)
        thetas = []
        for u_ref in key2:
            u2d = u_ref[...]
            theta = _bisect_theta(u2d, eq2, m)
            gt = eq2 & (u2d > theta)
            m = m - jnp.sum(gt.astype(jnp.int32))
            eq2 = eq2 & (u2d == theta)
            thetas.append(theta)

        idx2 = (lax.broadcasted_iota(jnp.int32, (BR, 128), 0) * 128
                + lax.broadcasted_iota(jnp.int32, (BR, 128), 1))

        def ibody(_, lohi):
            lo, hi = lohi
            mid = lo + ((hi - lo) >> 1)
            c = jnp.sum((eq2 & (idx2 < mid)).astype(jnp.int32))
            sel = c >= m
            return (jnp.where(sel, lo, mid + 1), jnp.where(sel, mid, hi))
        _, cut = lax.fori_loop(0, 15, ibody, (jnp.int32(0), jnp.int32(16384)))

        Mn = jnp.zeros((N, 1), bool)
        eqn = jnp.ones((N, 1), bool)
        for u_ref, theta in zip(keyn, thetas):
            un = u_ref[...]
            Mn = Mn | (eqn & (un > theta))
            eqn = eqn & (un == theta)
        idxn = lax.broadcasted_iota(jnp.int32, (N, 1), 0)
        Mn = Mn | (eqn & (idxn < cut))

        hm = (h_ref[...] * score_ref[...]) * Mn.astype(jnp.float32)
        hm_ref[...] = hm
        alive_ref[...] = Mn.astype(jnp.float32)
        rmax = jnp.max(jnp.where(Mn, hm, -jnp.inf), axis=0, keepdims=True)
        rmean = jnp.sum(hm, axis=0, keepdims=True) / k
        ro_ref[...] = jnp.concatenate([rmax, rmean], axis=1)
    return kern


def _pool(h, score, keyn_list, key2_list, k):
    nlevels = len(keyn_list)
    f = pl.pallas_call(
        _make_pool_kernel(k, nlevels),
        out_shape=(jax.ShapeDtypeStruct((N, H), jnp.float32),
                   jax.ShapeDtypeStruct((N, 1), jnp.float32),
                   jax.ShapeDtypeStruct((1, 2 * H), jnp.float32)),
    )
    return f(h, score, *keyn_list, *key2_list)


def _mlp_kernel(ro1_ref, ro2_ref, ro3_ref, wl1t_ref, bl1_ref, wl2t_ref,
                bl2_ref, wl3t_ref, bl3_ref, logits_ref, prob_ref, yhat_ref):
    z = ro1_ref[...] + ro2_ref[...] + ro3_ref[...]
    z = jnp.maximum(jnp.dot(z, wl1t_ref[...],
                            preferred_element_type=jnp.float32)
                    + bl1_ref[...], 0.0)
    z = jnp.maximum(jnp.dot(z, wl2t_ref[...],
                            preferred_element_type=jnp.float32)
                    + bl2_ref[...], 0.0)
    logits = jnp.dot(z, wl3t_ref[...],
                     preferred_element_type=jnp.float32) + bl3_ref[...]
    logits_ref[...] = logits
    mx = jnp.max(logits, axis=1, keepdims=True)
    ex = jnp.exp(logits - mx)
    prob_ref[...] = ex / jnp.sum(ex, axis=1, keepdims=True)
    yhat_ref[...] = (logits[:, 1:2] > logits[:, 0:1]).astype(jnp.int32)


def _mlp(ro1, ro2, ro3, wl1t, bl1, wl2t, bl2, wl3t, bl3):
    f = pl.pallas_call(
        _mlp_kernel,
        out_shape=(jax.ShapeDtypeStruct((1, 2), jnp.float32),
                   jax.ShapeDtypeStruct((1, 2), jnp.float32),
                   jax.ShapeDtypeStruct((1, 1), jnp.int32)),
    )
    return f(ro1, ro2, ro3, wl1t, bl1, wl2t, bl2, wl3t, bl3)


def _combine1(a0, a1, x, wrelt, wroott, b, p):
    f = pl.pallas_call(
        _combine1_kernel,
        out_shape=(jax.ShapeDtypeStruct((N, H), jnp.float32),
                   jax.ShapeDtypeStruct((N, 1), jnp.float32),
                   jax.ShapeDtypeStruct((N, 1), jnp.uint32)),
    )
    return f(a0, a1, x, wrelt, wroott, b, p)


def _combine23(a, hm, wrelt, wroott, b, p, alive):
    f = pl.pallas_call(
        _combine23_kernel,
        out_shape=(jax.ShapeDtypeStruct((N, H), jnp.float32),
                   jax.ShapeDtypeStruct((N, 1), jnp.float32),
                   jax.ShapeDtypeStruct((N, 1), jnp.uint32)),
    )
    return f(a, hm, wrelt, wroott, b, p, alive)


def _to2d(key):
    v = jnp.pad(jnp.reshape(key, (N,)), (0, NPAD - N))
    return jnp.reshape(v, (BR, 128))


def kernel(x, adj, Wrel1, Wroot1, b1, p1, Wrel2, Wroot2, b2, p2,
           Wrel3, Wroot3, b3, p3, Wl1, bl1, Wl2, bl2, Wl3, bl3):
    src = adj[0].astype(jnp.int32)
    dst = adj[1].astype(jnp.int32)
    zeros = jnp.zeros((AROWS, H), jnp.float32)

    src16 = jnp.reshape(src, (NS, NCHK, ECH))
    dst16 = jnp.reshape(dst, (NS, NCHK, ECH))

    a0, a1 = _edge_l1(x[:, 0:H], x[:, H:F_IN], src16, dst16, zeros)
    h, score, key1 = _combine1(a0, a1, x, Wrel1.T, Wroot1.T,
                               jnp.reshape(b1, (1, H)), jnp.reshape(p1, (1, H)))
    hm, alive, ro1 = _pool(h, score, [key1], [_to2d(key1)], K1)

    a = _edge_l23(hm, src16, dst16, zeros)
    h, score, key2 = _combine23(a, hm, Wrel2.T, Wroot2.T,
                                jnp.reshape(b2, (1, H)),
                                jnp.reshape(p2, (1, H)), alive)
    hm, alive, ro2 = _pool(h, score, [key2, key1],
                           [_to2d(key2), _to2d(key1)], K2)

    a = _edge_l23(hm, src16, dst16, zeros)
    h, score, key3 = _combine23(a, hm, Wrel3.T, Wroot3.T,
                                jnp.reshape(b3, (1, H)),
                                jnp.reshape(p3, (1, H)), alive)
    hm, alive, ro3 = _pool(h, score, [key3, key2, key1],
                           [_to2d(key3), _to2d(key2), _to2d(key1)], K3)

    logits, prob, yhat = _mlp(ro1, ro2, ro3, Wl1.T, jnp.reshape(bl1, (1, 128)),
                              Wl2.T, jnp.reshape(bl2, (1, 64)),
                              Wl3.T, jnp.reshape(bl3, (1, 2)))
    return (logits, prob, yhat)
